# Initial kernel scaffold; baseline (speedup 1.0000x reference)
import jax
import jax.numpy as jnp
from jax.experimental import pallas as pl


def kernel(bev_features, batch_centers):
    raise NotImplementedError("write your pallas kernel here")


# speedup vs baseline: 2.3374x; 2.3374x over previous
import functools

import jax
import jax.numpy as jnp
import numpy as np
from jax import lax
from jax.experimental import pallas as pl
from jax.experimental.pallas import tpu as pltpu
from jax.experimental.pallas import tpu_sc as plsc

B = 4
H = 180
W = 180
C = 256
N = 2500
NUM_POINT = 5
PC_START = (-54.0, -54.0)
VOXEL_SIZE = (0.075, 0.075)
OUT_STRIDE = 8

NROWS = B * N
K = 80
NCHUNKS = NROWS // K
NW = 32
TPW = (NCHUNKS + NW - 1) // NW
LANES = 16

_j = np.arange(NROWS)
_b = _j // N
_m = _j % N
_PERM = (_b * N + (_m % NUM_POINT) * (N // NUM_POINT) + _m // NUM_POINT).astype(
    np.int32
)


def _body(table_hbm, x_hbm, y_hbm, out_hbm, xv, yv, wbuf, idxbuf, gbuf, obuf, sem):
    wid = lax.axis_index("s") * 2 + lax.axis_index("c")

    for t in range(TPW):
        cid = t * NW + wid

        @pl.when(cid < NCHUNKS)
        def _chunk():
            row0 = cid * K
            pltpu.sync_copy(x_hbm.at[pl.ds(row0, K)], xv)
            pltpu.sync_copy(y_hbm.at[pl.ds(row0, K)], yv)

            for i in range(K // LANES):
                sl = pl.ds(i * LANES, LANES)
                x = xv[sl]
                y = yv[sl]
                x = (x - PC_START[0]) / VOXEL_SIZE[0] / OUT_STRIDE
                y = (y - PC_START[1]) / VOXEL_SIZE[1] / OUT_STRIDE
                x0 = x.astype(jnp.int32)
                y0 = y.astype(jnp.int32)
                x1 = jnp.minimum(x0 + 1, W - 1)
                x0 = jnp.minimum(jnp.maximum(x0, 0), W - 1)
                y1 = jnp.minimum(y0 + 1, H - 1)
                y0 = jnp.minimum(jnp.maximum(y0, 0), H - 1)
                x0f = x0.astype(jnp.float32)
                x1f = x1.astype(jnp.float32)
                y0f = y0.astype(jnp.float32)
                y1f = y1.astype(jnp.float32)
                wbuf[0, sl] = (x1f - x) * (y1f - y)
                wbuf[1, sl] = (x1f - x) * (y - y0f)
                wbuf[2, sl] = (x - x0f) * (y1f - y)
                wbuf[3, sl] = (x - x0f) * (y - y0f)
                g = row0 + i * LANES + lax.iota(jnp.int32, LANES)
                b = ((g - N) >> 31) + 1
                for bb in range(2, B):
                    b = b + (((g - bb * N) >> 31) + 1)
                base = b * (H * W)
                idxbuf[0, sl] = base + y0 * W + x0
                idxbuf[1, sl] = base + y1 * W + x0
                idxbuf[2, sl] = base + y0 * W + x1
                idxbuf[3, sl] = base + y1 * W + x1

            cps = [
                pltpu.async_copy(table_hbm.at[idxbuf.at[j]], gbuf.at[j], sem)
                for j in range(4)
            ]
            for cp in cps:
                cp.wait()

            def combine(k, carry):
                wa = wbuf[0, pl.ds(k, LANES)][0]
                wb = wbuf[1, pl.ds(k, LANES)][0]
                wc = wbuf[2, pl.ds(k, LANES)][0]
                wd = wbuf[3, pl.ds(k, LANES)][0]
                for c in range(C // LANES):
                    cs = pl.ds(c * LANES, LANES)
                    obuf[k, cs] = (
                        wa * gbuf[0, k, cs]
                        + wb * gbuf[1, k, cs]
                        + wc * gbuf[2, k, cs]
                        + wd * gbuf[3, k, cs]
                    )
                return carry

            lax.fori_loop(0, K, combine, 0)

            pltpu.sync_copy(obuf, out_hbm.at[pl.ds(row0, K)])


@functools.cache
def _sc_call():
    return pl.kernel(
        _body,
        out_type=jax.ShapeDtypeStruct((NROWS, C), jnp.float32),
        mesh=plsc.VectorSubcoreMesh(core_axis_name="c", subcore_axis_name="s"),
        scratch_types=[
            pltpu.VMEM((K,), jnp.float32),
            pltpu.VMEM((K,), jnp.float32),
            pltpu.VMEM((4, K + LANES), jnp.float32),
            pltpu.VMEM((4, K), jnp.int32),
            pltpu.VMEM((4, K, C), jnp.float32),
            pltpu.VMEM((K, C), jnp.float32),
            pltpu.SemaphoreType.DMA,
        ],
    )


@jax.jit
def kernel(bev_features, batch_centers):
    table = bev_features.reshape(B * H * W, C)
    perm = jnp.asarray(_PERM)
    xa = batch_centers[..., 0].reshape(-1)[perm]
    ya = batch_centers[..., 1].reshape(-1)[perm]
    out = _sc_call()(table, xa, ya)
    return out.reshape(B, N // NUM_POINT, NUM_POINT * C)


# speedup vs baseline: 5.6974x; 2.4375x over previous
import functools

import jax
import jax.numpy as jnp
import numpy as np
from jax import lax
from jax.experimental import pallas as pl
from jax.experimental.pallas import tpu as pltpu
from jax.experimental.pallas import tpu_sc as plsc

B = 4
H = 180
W = 180
C = 256
N = 2500
NUM_POINT = 5
PC_START = (-54.0, -54.0)
VOXEL_SIZE = (0.075, 0.075)
OUT_STRIDE = 8

NROWS = B * N
K = 50
NCHUNKS = NROWS // K
CPB = N // K
NW = 32
TPW = (NCHUNKS + NW - 1) // NW
LANES = 16
KPAD = 128
RY = 4
RL = 24
XCLAMP = 152
OROWS = K // NUM_POINT

_j = np.arange(NROWS)
_b = _j // N
_m = _j % N
_PERM = (_b * N + (_m % NUM_POINT) * (N // NUM_POINT) + _m // NUM_POINT).astype(
    np.int32
)


def _ge(v, c):
    return ((v - c) >> 31) + 1


def _body(bev_hbm, x_hbm, y_hbm, out_hbm, xv, yv, wbuf, ibuf, rbuf, sbuf, obuf):
    wid = lax.axis_index("s") * 2 + lax.axis_index("c")

    def chunk_body(t, carry):
        cid = t * NW + wid

        @pl.when(cid < NCHUNKS)
        def _chunk():
            b = _ge(cid, CPB) + _ge(cid, 2 * CPB) + _ge(cid, 3 * CPB)

            pltpu.sync_copy(x_hbm.at[pl.ds(cid * KPAD, KPAD)], xv)
            pltpu.sync_copy(y_hbm.at[pl.ds(cid * KPAD, KPAD)], yv)

            vymin = None
            for i in range(KPAD // LANES):
                sl = pl.ds(i * LANES, LANES)
                x = xv[sl]
                y = yv[sl]
                x = (x - PC_START[0]) / VOXEL_SIZE[0] / OUT_STRIDE
                y = (y - PC_START[1]) / VOXEL_SIZE[1] / OUT_STRIDE
                x0 = x.astype(jnp.int32)
                y0 = y.astype(jnp.int32)
                x1 = jnp.minimum(x0 + 1, W - 1)
                x0 = jnp.minimum(x0, W - 1)
                y1 = jnp.minimum(y0 + 1, H - 1)
                y0 = jnp.minimum(y0, H - 1)
                x0f = x0.astype(jnp.float32)
                x1f = x1.astype(jnp.float32)
                y0f = y0.astype(jnp.float32)
                y1f = y1.astype(jnp.float32)
                wbuf[0, sl] = (x1f - x) * (y1f - y)
                wbuf[1, sl] = (x1f - x) * (y - y0f)
                wbuf[2, sl] = (x - x0f) * (y1f - y)
                wbuf[3, sl] = (x - x0f) * (y - y0f)
                ibuf[0, sl] = y0
                ibuf[1, sl] = y1
                ibuf[2, sl] = x0
                ibuf[3, sl] = x1
                if vymin is None:
                    vymin, vymax, vxmin, vxmax = y0, y1, x0, x1
                else:
                    vymin = jnp.minimum(vymin, y0)
                    vymax = jnp.maximum(vymax, y1)
                    vxmin = jnp.minimum(vxmin, x0)
                    vxmax = jnp.maximum(vxmax, x1)

            def vreduce(v, op, sentinel):
                ibuf[4, pl.ds(0, LANES)] = v
                ibuf[4, pl.ds(LANES, LANES)] = jnp.full((LANES,), sentinel, jnp.int32)
                for sh in (8, 4, 2, 1):
                    a = ibuf[4, pl.ds(0, LANES)]
                    bb = ibuf[4, pl.ds(sh, LANES)]
                    ibuf[4, pl.ds(0, LANES)] = op(a, bb)
                return ibuf[4, pl.ds(0, LANES)][0]

            ymin = vreduce(vymin, jnp.minimum, 1000)
            ymax = vreduce(vymax, jnp.maximum, 0)
            xmin = vreduce(vxmin, jnp.minimum, 1000)
            xmax = vreduce(vxmax, jnp.maximum, 0)
            ybase = jnp.minimum(ymin, H - RY)
            xbase = pl.multiple_of(jnp.minimum(xmin & -8, XCLAMP), 8)
            fits = jnp.logical_and(ymax - ybase <= RY - 1, xmax - xbase <= RL - 1)

            def combine(r, p, yb, xb):
                k = r * NUM_POINT + p
                ks = pl.ds(k, LANES)
                ay = ibuf[0, ks][0] - yb
                ay1 = ibuf[1, ks][0] - yb
                ax = ibuf[2, ks][0] - xb
                ax1 = ibuf[3, ks][0] - xb
                w0 = wbuf[0, ks][0]
                w1 = wbuf[1, ks][0]
                w2 = wbuf[2, ks][0]
                w3 = wbuf[3, ks][0]
                for c in range(C // LANES):
                    cs = pl.ds(p * C + c * LANES, LANES)
                    bs = pl.ds(c * LANES, LANES)
                    obuf[r, cs] = (
                        w0 * rbuf[ay, ax, bs]
                        + w1 * rbuf[ay1, ax, bs]
                        + w2 * rbuf[ay, ax1, bs]
                        + w3 * rbuf[ay1, ax1, bs]
                    )

            @pl.when(fits)
            def _fast():
                pltpu.sync_copy(
                    bev_hbm.at[b, pl.ds(ybase, RY), pl.ds(xbase, RL), :], rbuf
                )

                def frow(r, carry2):
                    for p in range(NUM_POINT):
                        combine(r, p, ybase, xbase)
                    return carry2

                lax.fori_loop(0, OROWS, frow, 0)

            @pl.when(jnp.logical_not(fits))
            def _slow():
                def srow(r, carry2):
                    for p in range(NUM_POINT):
                        k = r * NUM_POINT + p
                        ks = pl.ds(k, LANES)
                        y0k = ibuf[0, ks][0]
                        y1k = ibuf[1, ks][0]
                        x0k = ibuf[2, ks][0]
                        x1k = ibuf[3, ks][0]
                        w0 = wbuf[0, ks][0]
                        w1 = wbuf[1, ks][0]
                        w2 = wbuf[2, ks][0]
                        w3 = wbuf[3, ks][0]
                        pltpu.sync_copy(
                            bev_hbm.at[b, pl.ds(y0k, 1), :, :], sbuf
                        )
                        for c in range(C // LANES):
                            cs = pl.ds(p * C + c * LANES, LANES)
                            bs = pl.ds(c * LANES, LANES)
                            obuf[r, cs] = (
                                w0 * sbuf[0, x0k, bs] + w2 * sbuf[0, x1k, bs]
                            )
                        pltpu.sync_copy(
                            bev_hbm.at[b, pl.ds(y1k, 1), :, :], sbuf
                        )
                        for c in range(C // LANES):
                            cs = pl.ds(p * C + c * LANES, LANES)
                            bs = pl.ds(c * LANES, LANES)
                            obuf[r, cs] = obuf[r, cs] + (
                                w1 * sbuf[0, x0k, bs] + w3 * sbuf[0, x1k, bs]
                            )
                    return carry2

                lax.fori_loop(0, OROWS, srow, 0)

            pltpu.sync_copy(obuf, out_hbm.at[cid])

        return carry

    lax.fori_loop(0, TPW, chunk_body, 0)


@functools.cache
def _sc_call():
    return pl.kernel(
        _body,
        out_type=jax.ShapeDtypeStruct((NCHUNKS, OROWS, NUM_POINT * C), jnp.float32),
        mesh=plsc.VectorSubcoreMesh(core_axis_name="c", subcore_axis_name="s"),
        compiler_params=pltpu.CompilerParams(use_tc_tiling_on_sc=True),
        scratch_types=[
            pltpu.VMEM((KPAD,), jnp.float32),
            pltpu.VMEM((KPAD,), jnp.float32),
            pltpu.VMEM((4, KPAD), jnp.float32),
            pltpu.VMEM((5, KPAD), jnp.int32),
            pltpu.VMEM((RY, RL, C), jnp.float32),
            pltpu.VMEM((1, W, C), jnp.float32),
            pltpu.VMEM((OROWS, NUM_POINT * C), jnp.float32),
        ],
    )


@jax.jit
def kernel(bev_features, batch_centers):
    perm = jnp.asarray(_PERM)
    xa = batch_centers[..., 0].reshape(-1)[perm]
    ya = batch_centers[..., 1].reshape(-1)[perm]
    x2 = jnp.pad(xa.reshape(NCHUNKS, K), ((0, 0), (0, KPAD - K))).reshape(-1)
    y2 = jnp.pad(ya.reshape(NCHUNKS, K), ((0, 0), (0, KPAD - K))).reshape(-1)
    out = _sc_call()(bev_features, x2, y2)
    return out.reshape(B, N // NUM_POINT, NUM_POINT * C)


# speedup vs baseline: 6.0321x; 1.0587x over previous
import functools

import jax
import jax.numpy as jnp
import numpy as np
from jax import lax
from jax.experimental import pallas as pl
from jax.experimental.pallas import tpu as pltpu
from jax.experimental.pallas import tpu_sc as plsc

B = 4
H = 180
W = 180
C = 256
N = 2500
NUM_POINT = 5
PC_START = (-54.0, -54.0)
VOXEL_SIZE = (0.075, 0.075)
OUT_STRIDE = 8

NROWS = B * N
K = 100
NCHUNKS = NROWS // K
CPB = N // K
NW = 32
TPW = (NCHUNKS + NW - 1) // NW
LANES = 16
PSTR = 20
WROWS = 128
RY = 4
RL = 24
XCLAMP = 152
OROWS = K // NUM_POINT
SEC = N // NUM_POINT
CRUN = 48


def _ge(v, c):
    return ((v - c) >> 31) + 1


def _body(
    bev_hbm, x_hbm, x4_hbm, y_hbm, y4_hbm, out_hbm,
    cxb, cyb, wbuf, ibuf, rbuf, sbuf, obuf, sem,
):
    wid = lax.axis_index("s") * 2 + lax.axis_index("c")

    def chunk_body(t, carry):
        cid = t * NW + wid

        @pl.when(cid < NCHUNKS)
        def _chunk():
            b = _ge(cid, CPB) + _ge(cid, 2 * CPB) + _ge(cid, 3 * CPB)
            cb = cid - b * CPB

            for p in range(NUM_POINT):
                start = b * N + p * SEC + cb * OROWS
                odd = (b + cb + p) & 1

                @pl.when(odd == 0)
                def _even(start=start, p=p):
                    s0 = pl.multiple_of(start, 8)
                    pltpu.async_copy(x_hbm.at[pl.ds(s0, CRUN)], cxb.at[p], sem)
                    pltpu.async_copy(y_hbm.at[pl.ds(s0, CRUN)], cyb.at[p], sem)

                @pl.when(odd == 1)
                def _odd(start=start, p=p):
                    s4 = pl.multiple_of(start - 4, 8)
                    pltpu.async_copy(x4_hbm.at[pl.ds(s4, CRUN)], cxb.at[p], sem)
                    pltpu.async_copy(y4_hbm.at[pl.ds(s4, CRUN)], cyb.at[p], sem)

            for p in range(NUM_POINT):
                pltpu.make_async_copy(x_hbm.at[pl.ds(0, CRUN)], cxb.at[p], sem).wait()
                pltpu.make_async_copy(y_hbm.at[pl.ds(0, CRUN)], cyb.at[p], sem).wait()

            vymin = None
            for p, g2 in [(p, g2) for p in range(NUM_POINT) for g2 in range(2)]:
                sl = pl.ds(p * PSTR + g2 * LANES, LANES)
                src = pl.ds(g2 * LANES, LANES)
                x = cxb[p, src]
                y = cyb[p, src]
                x = (x - PC_START[0]) / VOXEL_SIZE[0] / OUT_STRIDE
                y = (y - PC_START[1]) / VOXEL_SIZE[1] / OUT_STRIDE
                x0 = x.astype(jnp.int32)
                y0 = y.astype(jnp.int32)
                x1 = jnp.minimum(x0 + 1, W - 1)
                x0 = jnp.minimum(x0, W - 1)
                y1 = jnp.minimum(y0 + 1, H - 1)
                y0 = jnp.minimum(y0, H - 1)
                x0f = x0.astype(jnp.float32)
                x1f = x1.astype(jnp.float32)
                y0f = y0.astype(jnp.float32)
                y1f = y1.astype(jnp.float32)
                wbuf[0, sl] = (x1f - x) * (y1f - y)
                wbuf[1, sl] = (x1f - x) * (y - y0f)
                wbuf[2, sl] = (x - x0f) * (y1f - y)
                wbuf[3, sl] = (x - x0f) * (y - y0f)
                ibuf[0, sl] = y0
                ibuf[1, sl] = y1
                ibuf[2, sl] = x0
                ibuf[3, sl] = x1
                if vymin is None:
                    vymin, vymax, vxmin, vxmax = y0, y1, x0, x1
                else:
                    vymin = jnp.minimum(vymin, y0)
                    vymax = jnp.maximum(vymax, y1)
                    vxmin = jnp.minimum(vxmin, x0)
                    vxmax = jnp.maximum(vxmax, x1)

            def vreduce(v, op, sentinel):
                ibuf[4, pl.ds(0, LANES)] = v
                ibuf[4, pl.ds(LANES, LANES)] = jnp.full((LANES,), sentinel, jnp.int32)
                for sh in (8, 4, 2, 1):
                    a = ibuf[4, pl.ds(0, LANES)]
                    bb = ibuf[4, pl.ds(sh, LANES)]
                    ibuf[4, pl.ds(0, LANES)] = op(a, bb)
                return ibuf[4, pl.ds(0, LANES)][0]

            ymin = vreduce(vymin, jnp.minimum, 1000)
            ymax = vreduce(vymax, jnp.maximum, 0)
            xmin = vreduce(vxmin, jnp.minimum, 1000)
            xmax = vreduce(vxmax, jnp.maximum, 0)
            ybase = jnp.minimum(ymin, H - RY)
            xbase = pl.multiple_of(jnp.minimum(xmin & -8, XCLAMP), 8)
            fits = jnp.logical_and(ymax - ybase <= RY - 1, xmax - xbase <= RL - 1)

            @pl.when(fits)
            def _fast():
                pltpu.sync_copy(
                    bev_hbm.at[b, pl.ds(ybase, RY), pl.ds(xbase, RL), :], rbuf
                )

                def frow(r, carry2):
                    for p in range(NUM_POINT):
                        ks = pl.ds(p * PSTR + r, LANES)
                        ay = ibuf[0, ks][0] - ybase
                        ay1 = ibuf[1, ks][0] - ybase
                        ax = ibuf[2, ks][0] - xbase
                        ax1 = ibuf[3, ks][0] - xbase
                        w0 = wbuf[0, ks][0]
                        w1 = wbuf[1, ks][0]
                        w2 = wbuf[2, ks][0]
                        w3 = wbuf[3, ks][0]
                        for c in range(C // LANES):
                            cs = pl.ds(p * C + c * LANES, LANES)
                            bs = pl.ds(c * LANES, LANES)
                            obuf[r, cs] = (
                                w0 * rbuf[ay, ax, bs]
                                + w1 * rbuf[ay1, ax, bs]
                                + w2 * rbuf[ay, ax1, bs]
                                + w3 * rbuf[ay1, ax1, bs]
                            )
                    return carry2

                lax.fori_loop(0, OROWS, frow, 0)

            @pl.when(jnp.logical_not(fits))
            def _slow():
                def srow(r, carry2):
                    for p in range(NUM_POINT):
                        ks = pl.ds(p * PSTR + r, LANES)
                        y0k = ibuf[0, ks][0]
                        y1k = ibuf[1, ks][0]
                        x0k = ibuf[2, ks][0]
                        x1k = ibuf[3, ks][0]
                        w0 = wbuf[0, ks][0]
                        w1 = wbuf[1, ks][0]
                        w2 = wbuf[2, ks][0]
                        w3 = wbuf[3, ks][0]
                        for ch in range(2):
                            chs = pl.ds(ch * 128, 128)
                            pltpu.sync_copy(
                                bev_hbm.at[b, pl.ds(y0k, 1), :, chs], sbuf
                            )
                            for c in range(8):
                                cs = pl.ds(p * C + ch * 128 + c * LANES, LANES)
                                bs = pl.ds(c * LANES, LANES)
                                obuf[r, cs] = (
                                    w0 * sbuf[0, x0k, bs] + w2 * sbuf[0, x1k, bs]
                                )
                            pltpu.sync_copy(
                                bev_hbm.at[b, pl.ds(y1k, 1), :, chs], sbuf
                            )
                            for c in range(8):
                                cs = pl.ds(p * C + ch * 128 + c * LANES, LANES)
                                bs = pl.ds(c * LANES, LANES)
                                obuf[r, cs] = obuf[r, cs] + (
                                    w1 * sbuf[0, x0k, bs] + w3 * sbuf[0, x1k, bs]
                                )
                    return carry2

                lax.fori_loop(0, OROWS, srow, 0)

            pltpu.sync_copy(obuf, out_hbm.at[cid])

        return carry

    lax.fori_loop(0, TPW, chunk_body, 0)


@functools.cache
def _sc_call():
    return pl.kernel(
        _body,
        out_type=jax.ShapeDtypeStruct((NCHUNKS, OROWS, NUM_POINT * C), jnp.float32),
        mesh=plsc.VectorSubcoreMesh(core_axis_name="c", subcore_axis_name="s"),
        compiler_params=pltpu.CompilerParams(use_tc_tiling_on_sc=True),
        scratch_types=[
            pltpu.VMEM((NUM_POINT, CRUN), jnp.float32),
            pltpu.VMEM((NUM_POINT, CRUN), jnp.float32),
            pltpu.VMEM((4, WROWS), jnp.float32),
            pltpu.VMEM((5, WROWS), jnp.int32),
            pltpu.VMEM((RY, RL, C), jnp.float32),
            pltpu.VMEM((1, W, 128), jnp.float32),
            pltpu.VMEM((OROWS, NUM_POINT * C), jnp.float32),
            pltpu.SemaphoreType.DMA,
        ],
    )


@jax.jit
def kernel(bev_features, batch_centers):
    xa = jnp.pad(batch_centers[..., 0].reshape(-1), (0, 64))
    ya = jnp.pad(batch_centers[..., 1].reshape(-1), (0, 64))
    out = _sc_call()(bev_features, xa, xa[4:], ya, ya[4:])
    return out.reshape(B, N // NUM_POINT, NUM_POINT * C)


# speedup vs baseline: 7.6088x; 1.2614x over previous
import functools

import jax
import jax.numpy as jnp
import numpy as np
from jax import lax
from jax.experimental import pallas as pl
from jax.experimental.pallas import tpu as pltpu
from jax.experimental.pallas import tpu_sc as plsc

B = 4
H = 180
W = 180
C = 256
N = 2500
NUM_POINT = 5
PC_START = (-54.0, -54.0)
VOXEL_SIZE = (0.075, 0.075)
OUT_STRIDE = 8

RPC = 8
CPB = 63
NCHUNKS = B * CPB
NW = 32
CPS = 2 * CPB
TPW = (CPS + NW // 2 - 1) // (NW // 2)
LANES = 16
PSTR = RPC
WROWS = 128
RY = 4
RL = 24
XCLAMP = 152
SEC = N // NUM_POINT
CRUN = 16


def _ge(v, c):
    return ((v - c) >> 31) + 1


def _body(
    bev_hbm, x_hbm, x4_hbm, y_hbm, y4_hbm, out_hbm,
    cxb, cyb, wbuf, ibuf, rbuf, sbuf, slab, sem,
):
    core = lax.axis_index("c")
    lw = lax.axis_index("s")

    def chunk_body(t, carry):
        lcid = t * (NW // 2) + lw

        @pl.when(lcid < CPS)
        def _chunk():
            q = _ge(lcid, CPB)
            bb = core * 2 + q
            cb = lcid - q * CPB

            for p in range(NUM_POINT):
                start = bb * N + p * SEC + cb * RPC
                odd = (bb + p) & 1

                @pl.when(odd == 0)
                def _even(start=start, p=p):
                    s0 = pl.multiple_of(start, 8)
                    pltpu.async_copy(x_hbm.at[pl.ds(s0, CRUN)], cxb.at[p], sem)
                    pltpu.async_copy(y_hbm.at[pl.ds(s0, CRUN)], cyb.at[p], sem)

                @pl.when(odd == 1)
                def _odd(start=start, p=p):
                    s4 = pl.multiple_of(start - 4, 8)
                    pltpu.async_copy(x4_hbm.at[pl.ds(s4, CRUN)], cxb.at[p], sem)
                    pltpu.async_copy(y4_hbm.at[pl.ds(s4, CRUN)], cyb.at[p], sem)

            for p in range(NUM_POINT):
                pltpu.make_async_copy(x_hbm.at[pl.ds(0, CRUN)], cxb.at[p], sem).wait()
                pltpu.make_async_copy(y_hbm.at[pl.ds(0, CRUN)], cyb.at[p], sem).wait()

            vymin = None
            for p in range(NUM_POINT):
                sl = pl.ds(p * PSTR, LANES)
                x = cxb[p, pl.ds(0, LANES)]
                y = cyb[p, pl.ds(0, LANES)]
                x = (x - PC_START[0]) / VOXEL_SIZE[0] / OUT_STRIDE
                y = (y - PC_START[1]) / VOXEL_SIZE[1] / OUT_STRIDE
                x0 = x.astype(jnp.int32)
                y0 = y.astype(jnp.int32)
                x1 = jnp.minimum(x0 + 1, W - 1)
                x0 = jnp.minimum(x0, W - 1)
                y1 = jnp.minimum(y0 + 1, H - 1)
                y0 = jnp.minimum(y0, H - 1)
                x0f = x0.astype(jnp.float32)
                x1f = x1.astype(jnp.float32)
                y0f = y0.astype(jnp.float32)
                y1f = y1.astype(jnp.float32)
                wbuf[0, sl] = (x1f - x) * (y1f - y)
                wbuf[1, sl] = (x1f - x) * (y - y0f)
                wbuf[2, sl] = (x - x0f) * (y1f - y)
                wbuf[3, sl] = (x - x0f) * (y - y0f)
                ibuf[0, sl] = y0
                ibuf[1, sl] = y1
                ibuf[2, sl] = x0
                ibuf[3, sl] = x1
                if vymin is None:
                    vymin, vymax, vxmin, vxmax = y0, y1, x0, x1
                else:
                    vymin = jnp.minimum(vymin, y0)
                    vymax = jnp.maximum(vymax, y1)
                    vxmin = jnp.minimum(vxmin, x0)
                    vxmax = jnp.maximum(vxmax, x1)

            def vreduce(v, op, sentinel):
                ibuf[4, pl.ds(0, LANES)] = v
                ibuf[4, pl.ds(LANES, LANES)] = jnp.full((LANES,), sentinel, jnp.int32)
                for sh in (8, 4, 2, 1):
                    a = ibuf[4, pl.ds(0, LANES)]
                    bv = ibuf[4, pl.ds(sh, LANES)]
                    ibuf[4, pl.ds(0, LANES)] = op(a, bv)
                return ibuf[4, pl.ds(0, LANES)][0]

            ymin = vreduce(vymin, jnp.minimum, 1000)
            ymax = vreduce(vymax, jnp.maximum, 0)
            xmin = vreduce(vxmin, jnp.minimum, 1000)
            xmax = vreduce(vxmax, jnp.maximum, 0)
            ybase = jnp.minimum(ymin, H - RY)
            xbase = pl.multiple_of(jnp.minimum(xmin & -8, XCLAMP), 8)
            fits = jnp.logical_and(ymax - ybase <= RY - 1, xmax - xbase <= RL - 1)

            @pl.when(fits)
            def _fast():
                pltpu.sync_copy(
                    bev_hbm.at[bb, pl.ds(ybase, RY), pl.ds(xbase, RL), :], rbuf
                )

                def frow(r, carry2):
                    for p in range(NUM_POINT):
                        ks = pl.ds(p * PSTR + r, LANES)
                        ay = ibuf[0, ks][0] - ybase
                        ay1 = ibuf[1, ks][0] - ybase
                        ax = ibuf[2, ks][0] - xbase
                        ax1 = ibuf[3, ks][0] - xbase
                        w0 = wbuf[0, ks][0]
                        w1 = wbuf[1, ks][0]
                        w2 = wbuf[2, ks][0]
                        w3 = wbuf[3, ks][0]
                        for c in range(C // LANES):
                            cs = pl.ds(p * C + c * LANES, LANES)
                            bs = pl.ds(c * LANES, LANES)
                            slab[r, cs] = (
                                w0 * rbuf[ay, ax, bs]
                                + w1 * rbuf[ay1, ax, bs]
                                + w2 * rbuf[ay, ax1, bs]
                                + w3 * rbuf[ay1, ax1, bs]
                            )
                    return carry2

                lax.fori_loop(0, RPC, frow, 0)

            @pl.when(jnp.logical_not(fits))
            def _slow():
                def srow(r, carry2):
                    for p in range(NUM_POINT):
                        ks = pl.ds(p * PSTR + r, LANES)
                        y0k = ibuf[0, ks][0]
                        y1k = ibuf[1, ks][0]
                        x0k = ibuf[2, ks][0]
                        x1k = ibuf[3, ks][0]
                        w0 = wbuf[0, ks][0]
                        w1 = wbuf[1, ks][0]
                        w2 = wbuf[2, ks][0]
                        w3 = wbuf[3, ks][0]
                        for ch in range(2):
                            chs = pl.ds(ch * 128, 128)
                            pltpu.sync_copy(
                                bev_hbm.at[bb, pl.ds(y0k, 1), :, chs], sbuf
                            )
                            for c in range(8):
                                cs = pl.ds(p * C + ch * 128 + c * LANES, LANES)
                                bs = pl.ds(c * LANES, LANES)
                                slab[r, cs] = (
                                    w0 * sbuf[0, x0k, bs] + w2 * sbuf[0, x1k, bs]
                                )
                            pltpu.sync_copy(
                                bev_hbm.at[bb, pl.ds(y1k, 1), :, chs], sbuf
                            )
                            for c in range(8):
                                cs = pl.ds(p * C + ch * 128 + c * LANES, LANES)
                                bs = pl.ds(c * LANES, LANES)
                                slab[r, cs] = slab[r, cs] + (
                                    w1 * sbuf[0, x0k, bs] + w3 * sbuf[0, x1k, bs]
                                )
                    return carry2

                lax.fori_loop(0, RPC, srow, 0)

            ra = pl.multiple_of(cb * RPC, 8)
            pltpu.sync_copy(slab, out_hbm.at[bb, pl.ds(ra, RPC), :])

        return carry

    lax.fori_loop(0, TPW, chunk_body, 0)


@functools.cache
def _sc_call():
    return pl.kernel(
        _body,
        out_type=jax.ShapeDtypeStruct((B, N // NUM_POINT, NUM_POINT * C), jnp.float32),
        mesh=plsc.VectorSubcoreMesh(core_axis_name="c", subcore_axis_name="s"),
        compiler_params=pltpu.CompilerParams(use_tc_tiling_on_sc=True),
        scratch_types=[
            pltpu.VMEM((NUM_POINT, CRUN), jnp.float32),
            pltpu.VMEM((NUM_POINT, CRUN), jnp.float32),
            pltpu.VMEM((4, WROWS), jnp.float32),
            pltpu.VMEM((5, WROWS), jnp.int32),
            pltpu.VMEM((RY, RL, C), jnp.float32),
            pltpu.VMEM((1, W, 128), jnp.float32),
            pltpu.VMEM((RPC, NUM_POINT * C), jnp.float32),
            pltpu.SemaphoreType.DMA,
        ],
    )


@jax.jit
def kernel(bev_features, batch_centers):
    xa = jnp.pad(batch_centers[..., 0].reshape(-1), (0, 64))
    ya = jnp.pad(batch_centers[..., 1].reshape(-1), (0, 64))
    return _sc_call()(bev_features, xa, xa[4:], ya, ya[4:])


# speedup vs baseline: 7.6966x; 1.0115x over previous
import functools

import jax
import jax.numpy as jnp
import numpy as np
from jax import lax
from jax.experimental import pallas as pl
from jax.experimental.pallas import tpu as pltpu
from jax.experimental.pallas import tpu_sc as plsc

B = 4
H = 180
W = 180
C = 256
N = 2500
NUM_POINT = 5
PC_START = (-54.0, -54.0)
VOXEL_SIZE = (0.075, 0.075)
OUT_STRIDE = 8

RPC = 8
CPB = 63
NCHUNKS = B * CPB
NW = 32
CPS = 2 * CPB
TPW = (CPS + NW // 2 - 1) // (NW // 2)
LANES = 16
PSTR = RPC
WROWS = 128
RY = 4
RL = 24
XCLAMP = 152
SEC = N // NUM_POINT
CRUN = 24
YOFF = B * N + 16


def _ge(v, c):
    return ((v - c) >> 31) + 1


def _body(
    bev_hbm, xy_hbm, out_hbm,
    cxb, cyb, wbuf, ibuf, rbuf, sbuf, slab, sem,
):
    core = lax.axis_index("c")
    lw = lax.axis_index("s")

    def chunk_body(t, carry):
        lcid = t * (NW // 2) + lw

        @pl.when(lcid < CPS)
        def _chunk():
            q = _ge(lcid, CPB)
            bb = core * 2 + q
            cb = lcid - q * CPB

            for p in range(NUM_POINT):
                start = bb * N + p * SEC + cb * RPC
                s0 = pl.multiple_of(start & -8, 8)
                sy = pl.multiple_of((YOFF + start) & -8, 8)
                pltpu.async_copy(xy_hbm.at[pl.ds(s0, CRUN)], cxb.at[p], sem)
                pltpu.async_copy(xy_hbm.at[pl.ds(sy, CRUN)], cyb.at[p], sem)

            for p in range(NUM_POINT):
                pltpu.make_async_copy(xy_hbm.at[pl.ds(0, CRUN)], cxb.at[p], sem).wait()
                pltpu.make_async_copy(xy_hbm.at[pl.ds(0, CRUN)], cyb.at[p], sem).wait()

            vymin = None
            for p in range(NUM_POINT):
                sl = pl.ds(p * PSTR, LANES)
                start = bb * N + p * SEC + cb * RPC
                hf = ((start >> 2) & 1).astype(jnp.float32)
                x = cxb[p, pl.ds(0, LANES)] * (1.0 - hf) + cxb[p, pl.ds(4, LANES)] * hf
                y = cyb[p, pl.ds(0, LANES)] * (1.0 - hf) + cyb[p, pl.ds(4, LANES)] * hf
                x = (x - PC_START[0]) / VOXEL_SIZE[0] / OUT_STRIDE
                y = (y - PC_START[1]) / VOXEL_SIZE[1] / OUT_STRIDE
                x0 = x.astype(jnp.int32)
                y0 = y.astype(jnp.int32)
                x0 = jnp.minimum(jnp.maximum(x0, 0), W - 1)
                y0 = jnp.minimum(jnp.maximum(y0, 0), H - 1)
                x1 = jnp.minimum(x0 + 1, W - 1)
                y1 = jnp.minimum(y0 + 1, H - 1)
                x0f = x0.astype(jnp.float32)
                x1f = x1.astype(jnp.float32)
                y0f = y0.astype(jnp.float32)
                y1f = y1.astype(jnp.float32)
                wbuf[0, sl] = (x1f - x) * (y1f - y)
                wbuf[1, sl] = (x1f - x) * (y - y0f)
                wbuf[2, sl] = (x - x0f) * (y1f - y)
                wbuf[3, sl] = (x - x0f) * (y - y0f)
                ibuf[0, sl] = y0
                ibuf[1, sl] = y1
                ibuf[2, sl] = x0
                ibuf[3, sl] = x1
                if vymin is None:
                    vymin, vymax, vxmin, vxmax = y0, y1, x0, x1
                else:
                    vymin = jnp.minimum(vymin, y0)
                    vymax = jnp.maximum(vymax, y1)
                    vxmin = jnp.minimum(vxmin, x0)
                    vxmax = jnp.maximum(vxmax, x1)

            def vreduce(v, op, sentinel):
                ibuf[4, pl.ds(0, LANES)] = v
                ibuf[4, pl.ds(LANES, LANES)] = jnp.full((LANES,), sentinel, jnp.int32)
                for sh in (8, 4, 2, 1):
                    a = ibuf[4, pl.ds(0, LANES)]
                    bv = ibuf[4, pl.ds(sh, LANES)]
                    ibuf[4, pl.ds(0, LANES)] = op(a, bv)
                return ibuf[4, pl.ds(0, LANES)][0]

            ymin = vreduce(vymin, jnp.minimum, 1000)
            ymax = vreduce(vymax, jnp.maximum, 0)
            xmin = vreduce(vxmin, jnp.minimum, 1000)
            xmax = vreduce(vxmax, jnp.maximum, 0)
            ybase = jnp.minimum(ymin, H - RY)
            xbase = pl.multiple_of(jnp.minimum(xmin & -8, XCLAMP), 8)
            fits = jnp.logical_and(ymax - ybase <= RY - 1, xmax - xbase <= RL - 1)

            @pl.when(fits)
            def _fast():
                pltpu.sync_copy(
                    bev_hbm.at[bb, pl.ds(ybase, RY), pl.ds(xbase, RL), :], rbuf
                )

                def frow(r, carry2):
                    for p in range(NUM_POINT):
                        ks = pl.ds(p * PSTR + r, LANES)
                        ay = ibuf[0, ks][0] - ybase
                        ay1 = ibuf[1, ks][0] - ybase
                        ax = ibuf[2, ks][0] - xbase
                        ax1 = ibuf[3, ks][0] - xbase
                        w0 = wbuf[0, ks][0]
                        w1 = wbuf[1, ks][0]
                        w2 = wbuf[2, ks][0]
                        w3 = wbuf[3, ks][0]
                        for c in range(C // LANES):
                            cs = pl.ds(p * C + c * LANES, LANES)
                            bs = pl.ds(c * LANES, LANES)
                            slab[r, cs] = (
                                w0 * rbuf[ay, ax, bs]
                                + w1 * rbuf[ay1, ax, bs]
                                + w2 * rbuf[ay, ax1, bs]
                                + w3 * rbuf[ay1, ax1, bs]
                            )
                    return carry2

                lax.fori_loop(0, RPC, frow, 0)

            @pl.when(jnp.logical_not(fits))
            def _slow():
                def srow(r, carry2):
                    for p in range(NUM_POINT):
                        ks = pl.ds(p * PSTR + r, LANES)
                        y0k = ibuf[0, ks][0]
                        y1k = ibuf[1, ks][0]
                        x0k = ibuf[2, ks][0]
                        x1k = ibuf[3, ks][0]
                        w0 = wbuf[0, ks][0]
                        w1 = wbuf[1, ks][0]
                        w2 = wbuf[2, ks][0]
                        w3 = wbuf[3, ks][0]
                        for ch in range(2):
                            chs = pl.ds(ch * 128, 128)
                            pltpu.sync_copy(
                                bev_hbm.at[bb, pl.ds(y0k, 1), :, chs], sbuf
                            )
                            for c in range(8):
                                cs = pl.ds(p * C + ch * 128 + c * LANES, LANES)
                                bs = pl.ds(c * LANES, LANES)
                                slab[r, cs] = (
                                    w0 * sbuf[0, x0k, bs] + w2 * sbuf[0, x1k, bs]
                                )
                            pltpu.sync_copy(
                                bev_hbm.at[bb, pl.ds(y1k, 1), :, chs], sbuf
                            )
                            for c in range(8):
                                cs = pl.ds(p * C + ch * 128 + c * LANES, LANES)
                                bs = pl.ds(c * LANES, LANES)
                                slab[r, cs] = slab[r, cs] + (
                                    w1 * sbuf[0, x0k, bs] + w3 * sbuf[0, x1k, bs]
                                )
                    return carry2

                lax.fori_loop(0, RPC, srow, 0)

            ra = pl.multiple_of(cb * RPC, 8)
            pltpu.sync_copy(slab, out_hbm.at[bb, pl.ds(ra, RPC), :])

        return carry

    lax.fori_loop(0, TPW, chunk_body, 0)


@functools.cache
def _sc_call():
    return pl.kernel(
        _body,
        out_type=jax.ShapeDtypeStruct((B, N // NUM_POINT, NUM_POINT * C), jnp.float32),
        mesh=plsc.VectorSubcoreMesh(core_axis_name="c", subcore_axis_name="s"),
        compiler_params=pltpu.CompilerParams(use_tc_tiling_on_sc=True),
        scratch_types=[
            pltpu.VMEM((NUM_POINT, CRUN), jnp.float32),
            pltpu.VMEM((NUM_POINT, CRUN), jnp.float32),
            pltpu.VMEM((4, WROWS), jnp.float32),
            pltpu.VMEM((5, WROWS), jnp.int32),
            pltpu.VMEM((RY, RL, C), jnp.float32),
            pltpu.VMEM((1, W, 128), jnp.float32),
            pltpu.VMEM((RPC, NUM_POINT * C), jnp.float32),
            pltpu.SemaphoreType.DMA,
        ],
    )


@jax.jit
def kernel(bev_features, batch_centers):
    xy = jnp.concatenate(
        [
            jnp.pad(batch_centers[..., 0].reshape(-1), (0, 16)),
            jnp.pad(batch_centers[..., 1].reshape(-1), (0, 16)),
        ]
    )
    return _sc_call()(bev_features, xy)
